# baseline (speedup 1.0000x reference)
import jax
import jax.numpy as jnp
from jax.experimental import pallas as pl
from jax.experimental.pallas import tpu as pltpu

_TB = 512


def _body(w_ref, adj_ref, g50_ref, g4_ref, bq_ref, bs_ref,
          fp_ref, qos_ref, fh_ref,
          trust_ref, cons_ref, supp_ref, stab_ref):
    f32 = jnp.float32
    fh = fh_ref[...]
    g50 = g50_ref[...]
    s1 = jnp.dot(fh, g50, preferred_element_type=f32)
    s2 = jnp.dot(fh * fh, g50, preferred_element_type=f32)
    inv_w = 1.0 / fh.shape[1] * g50.shape[1]
    mean = s1 * inv_w
    var = s2 * inv_w - mean * mean
    stab = 1.0 / (1.0 + var)

    qn = (qos_ref[...] - bq_ref[...]) * bs_ref[...]
    nsq = jnp.dot(qn * qn, g4_ref[...], preferred_element_type=f32)
    supp = jax.nn.sigmoid(jnp.sqrt(nsq))

    fp = fp_ref[...]
    m = (adj_ref[...] > 0).astype(f32)
    counts = jnp.sum(m, axis=0, keepdims=True)
    pf = (fp > 0.5).astype(f32)
    mpf = jnp.dot(pf, m, preferred_element_type=f32) / jnp.maximum(counts, 1.0)
    consistent = (mpf <= fp + 0.3).astype(f32)
    cons = jnp.where(counts > 0, consistent, 1.0)

    w1 = w_ref[0]
    w2 = w_ref[1]
    w3 = w_ref[2]
    trust_ref[...] = w1 * cons + w2 * supp + w3 * stab
    cons_ref[...] = cons
    supp_ref[...] = supp
    stab_ref[...] = stab


def kernel(fault_probs, qos_observations, fault_history, adjacency_matrix,
           gamma1, gamma2, gamma3, baseline_qos, baseline_std):
    B, N, W = fault_history.shape
    Q = qos_observations.shape[-1]
    fp2 = fault_probs.reshape(B, N)
    qos2 = qos_observations.reshape(B, N * Q)
    fh2 = fault_history.reshape(B, N * W)

    gsum = gamma1 + gamma2 + gamma3 + 1e-8
    w = jnp.stack([gamma1 / gsum, gamma2 / gsum, gamma3 / gsum]).astype(jnp.float32)

    nodes = jnp.arange(N, dtype=jnp.int32)
    g50 = (jnp.arange(N * W, dtype=jnp.int32)[:, None] // W == nodes[None, :]).astype(jnp.float32)
    g4 = (jnp.arange(N * Q, dtype=jnp.int32)[:, None] // Q == nodes[None, :]).astype(jnp.float32)
    bq = jnp.tile(baseline_qos, N)[None, :]
    bs = jnp.tile(1.0 / (baseline_std + 1e-8), N)[None, :]

    grid = (B // _TB,)
    const = lambda shape: pl.BlockSpec(shape, lambda i: (0, 0))
    row = lambda cols: pl.BlockSpec((_TB, cols), lambda i: (i, 0))
    out_shape = jax.ShapeDtypeStruct((B, N), jnp.float32)

    trust, cons, supp, stab = pl.pallas_call(
        _body,
        grid=grid,
        in_specs=[
            pl.BlockSpec(memory_space=pltpu.SMEM),
            const((N, N)),
            const((N * W, N)),
            const((N * Q, N)),
            pl.BlockSpec((1, N * Q), lambda i: (0, 0)),
            pl.BlockSpec((1, N * Q), lambda i: (0, 0)),
            row(N),
            row(N * Q),
            row(N * W),
        ],
        out_specs=[row(N), row(N), row(N), row(N)],
        out_shape=[out_shape] * 4,
        compiler_params=pltpu.CompilerParams(
            dimension_semantics=("arbitrary",),
        ),
    )(w, adjacency_matrix, g50, g4, bq, bs, fp2, qos2, fh2)

    shape3 = (B, N, 1)
    return (trust.reshape(shape3), cons.reshape(shape3),
            supp.reshape(shape3), stab.reshape(shape3))


# speedup vs baseline: 5.3113x; 5.3113x over previous
import jax
import jax.numpy as jnp
from jax import lax
from jax.experimental import pallas as pl
from jax.experimental.pallas import tpu as pltpu

_BB = 1024


def _body(w_ref, adj_ref, bq_ref, bs_ref, fp_ref, qos_ref, fh_ref,
          trust_ref, cons_ref, supp_ref, stab_ref):
    f32 = jnp.float32
    W = fh_ref.shape[0] // adj_ref.shape[0]
    N = adj_ref.shape[0]
    blk = fh_ref[...].reshape(W, N, fh_ref.shape[1])
    v = blk[0]
    s1 = v
    s2 = v * v
    for wi in range(1, W):
        v = blk[wi]
        s1 = s1 + v
        s2 = s2 + v * v
    inv_w = f32(1.0 / W)
    mean = s1 * inv_w
    var = s2 * inv_w - mean * mean
    stab = 1.0 / (1.0 + var)

    qn = (qos_ref[...] - bq_ref[...]) * bs_ref[...]
    nsq = jnp.sum(qn * qn, axis=1)
    supp = jax.nn.sigmoid(jnp.sqrt(nsq))

    fp = fp_ref[...]
    m = (adj_ref[...] > 0).astype(f32)
    counts = jnp.sum(m, axis=0, keepdims=True)
    pf = (fp > 0.5).astype(f32)
    numer = lax.dot_general(m, pf, (((0,), (0,)), ((), ())),
                            preferred_element_type=f32)
    mpf = numer / jnp.maximum(counts.T, 1.0)
    consistent = (mpf <= fp + 0.3).astype(f32)
    cons = jnp.where(counts.T > 0, consistent, 1.0)

    w1 = w_ref[0]
    w2 = w_ref[1]
    w3 = w_ref[2]
    trust_ref[...] = w1 * cons + w2 * supp + w3 * stab
    cons_ref[...] = cons
    supp_ref[...] = supp
    stab_ref[...] = stab


def kernel(fault_probs, qos_observations, fault_history, adjacency_matrix,
           gamma1, gamma2, gamma3, baseline_qos, baseline_std):
    B, N, W = fault_history.shape
    Q = qos_observations.shape[-1]
    fh_lin = fault_history.transpose(2, 1, 0).reshape(W * N, B)
    qos3 = qos_observations.transpose(1, 2, 0)
    fp2 = fault_probs.transpose(1, 2, 0).reshape(N, B)

    gsum = gamma1 + gamma2 + gamma3 + 1e-8
    w = jnp.stack([gamma1 / gsum, gamma2 / gsum, gamma3 / gsum]).astype(jnp.float32)
    bq = baseline_qos[None, :, None]
    bs = (1.0 / (baseline_std + 1e-8))[None, :, None]

    grid = (B // _BB,)
    out_shape = jax.ShapeDtypeStruct((N, B), jnp.float32)

    trust, cons, supp, stab = pl.pallas_call(
        _body,
        grid=grid,
        in_specs=[
            pl.BlockSpec(memory_space=pltpu.SMEM),
            pl.BlockSpec((N, N), lambda j: (0, 0)),
            pl.BlockSpec((1, Q, 1), lambda j: (0, 0, 0)),
            pl.BlockSpec((1, Q, 1), lambda j: (0, 0, 0)),
            pl.BlockSpec((N, _BB), lambda j: (0, j)),
            pl.BlockSpec((N, Q, _BB), lambda j: (0, 0, j)),
            pl.BlockSpec((W * N, _BB), lambda j: (0, j)),
        ],
        out_specs=[pl.BlockSpec((N, _BB), lambda j: (0, j))] * 4,
        out_shape=[out_shape] * 4,
        compiler_params=pltpu.CompilerParams(
            dimension_semantics=("arbitrary",),
        ),
    )(w, adjacency_matrix, bq, bs, fp2, qos3, fh_lin)

    def back(a):
        return a.T[:, :, None]

    return back(trust), back(cons), back(supp), back(stab)


# speedup vs baseline: 5.7859x; 1.0894x over previous
import jax
import jax.numpy as jnp
from jax import lax
from jax.experimental import pallas as pl
from jax.experimental.pallas import tpu as pltpu

_BB = 1024


def _body(w_ref, adj_ref, bq_ref, bs_ref, fp_ref, qos_ref, fh_ref,
          trust_ref, cons_ref, supp_ref, stab_ref):
    f32 = jnp.float32
    N = adj_ref.shape[0]
    W = fh_ref.shape[0] // N
    S = 8

    fp = fp_ref[...]
    m = (adj_ref[...] > 0).astype(f32)
    counts = jnp.sum(m, axis=0, keepdims=True)
    pf = (fp > 0.5).astype(f32)
    numer = lax.dot_general(m, pf, (((0,), (0,)), ((), ())),
                            preferred_element_type=f32)
    mpf = numer / jnp.maximum(counts.T, 1.0)
    consistent = (mpf <= fp + 0.3).astype(f32)
    cons = jnp.where(counts.T > 0, consistent, 1.0)
    cons_ref[...] = cons

    w1 = w_ref[0]
    w2 = w_ref[1]
    w3 = w_ref[2]
    inv_w = f32(1.0 / W)

    for nb in range(N // S):
        lo = nb * S
        v = fh_ref[pl.ds(lo, S), :]
        a1 = v
        a2 = v * v
        for wi in range(1, W):
            v = fh_ref[pl.ds(wi * N + lo, S), :]
            a1 = a1 + v
            a2 = a2 + v * v
        mean = a1 * inv_w
        var = a2 * inv_w - mean * mean
        stab = 1.0 / (1.0 + var)

        qn = (qos_ref[pl.ds(lo, S)] - bq_ref[...]) * bs_ref[...]
        nsq = jnp.sum(qn * qn, axis=1)
        supp = jax.nn.sigmoid(jnp.sqrt(nsq))

        cs = cons[lo:lo + S, :]
        trust_ref[pl.ds(lo, S), :] = w1 * cs + w2 * supp + w3 * stab
        supp_ref[pl.ds(lo, S), :] = supp
        stab_ref[pl.ds(lo, S), :] = stab


def kernel(fault_probs, qos_observations, fault_history, adjacency_matrix,
           gamma1, gamma2, gamma3, baseline_qos, baseline_std):
    B, N, W = fault_history.shape
    Q = qos_observations.shape[-1]
    fh_lin = fault_history.transpose(2, 1, 0).reshape(W * N, B)
    qos3 = qos_observations.transpose(1, 2, 0)
    fp2 = fault_probs.transpose(1, 2, 0).reshape(N, B)

    gsum = gamma1 + gamma2 + gamma3 + 1e-8
    w = jnp.stack([gamma1 / gsum, gamma2 / gsum, gamma3 / gsum]).astype(jnp.float32)
    bq = baseline_qos[None, :, None]
    bs = (1.0 / (baseline_std + 1e-8))[None, :, None]

    grid = (B // _BB,)
    out_shape = jax.ShapeDtypeStruct((N, B), jnp.float32)

    trust, cons, supp, stab = pl.pallas_call(
        _body,
        grid=grid,
        in_specs=[
            pl.BlockSpec(memory_space=pltpu.SMEM),
            pl.BlockSpec((N, N), lambda j: (0, 0)),
            pl.BlockSpec((1, Q, 1), lambda j: (0, 0, 0)),
            pl.BlockSpec((1, Q, 1), lambda j: (0, 0, 0)),
            pl.BlockSpec((N, _BB), lambda j: (0, j)),
            pl.BlockSpec((N, Q, _BB), lambda j: (0, 0, j)),
            pl.BlockSpec((W * N, _BB), lambda j: (0, j)),
        ],
        out_specs=[pl.BlockSpec((N, _BB), lambda j: (0, j))] * 4,
        out_shape=[out_shape] * 4,
        compiler_params=pltpu.CompilerParams(
            dimension_semantics=("arbitrary",),
        ),
    )(w, adjacency_matrix, bq, bs, fp2, qos3, fh_lin)

    def back(a):
        return a.T[:, :, None]

    return back(trust), back(cons), back(supp), back(stab)


# speedup vs baseline: 5.9346x; 1.0257x over previous
import jax
import jax.numpy as jnp
from jax import lax
from jax.experimental import pallas as pl
from jax.experimental.pallas import tpu as pltpu

_BB = 2048


def _body(w_ref, adj_ref, bq_ref, bs_ref, fp_ref, qos_ref, fh_ref,
          trust_ref, cons_ref, supp_ref, stab_ref):
    f32 = jnp.float32
    N = adj_ref.shape[0]
    W = fh_ref.shape[0] // N
    S = 8

    fp = fp_ref[...]
    m = (adj_ref[...] > 0).astype(f32)
    counts = jnp.sum(m, axis=0, keepdims=True)
    pf = (fp > 0.5).astype(f32)
    numer = lax.dot_general(m, pf, (((0,), (0,)), ((), ())),
                            preferred_element_type=f32)
    mpf = numer / jnp.maximum(counts.T, 1.0)
    consistent = (mpf <= fp + 0.3).astype(f32)
    cons = jnp.where(counts.T > 0, consistent, 1.0)
    cons_ref[...] = cons

    w1 = w_ref[0]
    w2 = w_ref[1]
    w3 = w_ref[2]
    inv_w = f32(1.0 / W)

    for nb in range(N // S):
        lo = nb * S
        v = fh_ref[pl.ds(lo, S), :]
        a1 = v
        a2 = v * v
        for wi in range(1, W):
            v = fh_ref[pl.ds(wi * N + lo, S), :]
            a1 = a1 + v
            a2 = a2 + v * v
        mean = a1 * inv_w
        var = a2 * inv_w - mean * mean
        stab = 1.0 / (1.0 + var)

        qn = (qos_ref[pl.ds(lo, S)] - bq_ref[...]) * bs_ref[...]
        nsq = jnp.sum(qn * qn, axis=1)
        supp = jax.nn.sigmoid(jnp.sqrt(nsq))

        cs = cons[lo:lo + S, :]
        trust_ref[pl.ds(lo, S), :] = w1 * cs + w2 * supp + w3 * stab
        supp_ref[pl.ds(lo, S), :] = supp
        stab_ref[pl.ds(lo, S), :] = stab


def kernel(fault_probs, qos_observations, fault_history, adjacency_matrix,
           gamma1, gamma2, gamma3, baseline_qos, baseline_std):
    B, N, W = fault_history.shape
    Q = qos_observations.shape[-1]
    fh_lin = fault_history.transpose(2, 1, 0).reshape(W * N, B)
    qos3 = qos_observations.transpose(1, 2, 0)
    fp2 = fault_probs.transpose(1, 2, 0).reshape(N, B)

    gsum = gamma1 + gamma2 + gamma3 + 1e-8
    w = jnp.stack([gamma1 / gsum, gamma2 / gsum, gamma3 / gsum]).astype(jnp.float32)
    bq = baseline_qos[None, :, None]
    bs = (1.0 / (baseline_std + 1e-8))[None, :, None]

    grid = (B // _BB,)
    out_shape = jax.ShapeDtypeStruct((N, B), jnp.float32)

    trust, cons, supp, stab = pl.pallas_call(
        _body,
        grid=grid,
        in_specs=[
            pl.BlockSpec(memory_space=pltpu.SMEM),
            pl.BlockSpec((N, N), lambda j: (0, 0)),
            pl.BlockSpec((1, Q, 1), lambda j: (0, 0, 0)),
            pl.BlockSpec((1, Q, 1), lambda j: (0, 0, 0)),
            pl.BlockSpec((N, _BB), lambda j: (0, j)),
            pl.BlockSpec((N, Q, _BB), lambda j: (0, 0, j)),
            pl.BlockSpec((W * N, _BB), lambda j: (0, j)),
        ],
        out_specs=[pl.BlockSpec((N, _BB), lambda j: (0, j))] * 4,
        out_shape=[out_shape] * 4,
        compiler_params=pltpu.CompilerParams(
            dimension_semantics=("arbitrary",),
        ),
    )(w, adjacency_matrix, bq, bs, fp2, qos3, fh_lin)

    def back(a):
        return a.T[:, :, None]

    return back(trust), back(cons), back(supp), back(stab)
